# initial kernel scaffold (unmeasured)
import jax
import jax.numpy as jnp
from jax import lax
from jax.experimental import pallas as pl
from jax.experimental.pallas import tpu as pltpu

NDEV = 8
NTOK = 2048
D = 1024
NLOC = 8
NEXP = NDEV * NLOC
CAP = 64
BLK = NLOC * CAP


def kernel(x, router_W, route_idx, expert_W, shared_W):
    def body(x_ref, rw_ref, idx_ref, ew_ref, sw_ref, out_ref,
             disp_ref, recv_ref, ret_ref, retbuf_ref,
             d_send, d_recv, r_send, r_recv):
        my = lax.axis_index("i")

        barrier = pltpu.get_barrier_semaphore()
        for o in range(1, NDEV):
            pl.semaphore_signal(
                barrier, inc=1,
                device_id=((my + o) % NDEV,),
                device_id_type=pl.DeviceIdType.MESH,
            )
        pl.semaphore_wait(barrier, NDEV - 1)

        xf = x_ref[...]
        scores = jnp.dot(xf, rw_ref[...], preferred_element_type=jnp.float32)
        smax = jnp.max(scores, axis=1, keepdims=True)
        ex = jnp.exp(scores - smax)
        probs = ex / jnp.sum(ex, axis=1, keepdims=True)
        e = idx_ref[...]
        eoh = (lax.broadcasted_iota(jnp.int32, (NTOK, NEXP), 1) == e)
        eohf = eoh.astype(jnp.float32)
        psel = jnp.sum(probs * eohf, axis=1, keepdims=True)
        xs = (xf * psel).astype(jnp.bfloat16)

        c = eohf
        shift = 1
        while shift < NTOK:
            c = c + jnp.concatenate(
                [jnp.zeros((shift, NEXP), jnp.float32), c[:-shift]], axis=0)
            shift *= 2
        slot = jnp.sum((c - eohf) * eohf, axis=1, keepdims=True).astype(jnp.int32)
        dslot = jnp.where(slot < CAP, e * CAP + slot, jnp.int32(1 << 20))

        def onehot_T(o):
            j = (my + o) % NDEV
            loc = dslot - j * BLK
            return (loc == lax.broadcasted_iota(jnp.int32, (NTOK, BLK), 1)
                    ).astype(jnp.bfloat16)

        for o in range(NDEV):
            t = onehot_T(o)
            blk = lax.dot_general(t, xs, (((0,), (0,)), ((), ())),
                                  preferred_element_type=jnp.float32)
            disp_ref[o] = blk.astype(jnp.bfloat16)

        loc_blk = disp_ref[0]
        for k in range(NLOC):
            recv_ref[k, 0] = loc_blk[k * CAP:(k + 1) * CAP]

        dispatch = []
        for o in range(1, NDEV):
            for k in range(NLOC):
                rdma = pltpu.make_async_remote_copy(
                    src_ref=disp_ref.at[o, pl.ds(k * CAP, CAP)],
                    dst_ref=recv_ref.at[k, o],
                    send_sem=d_send.at[k, o],
                    recv_sem=d_recv.at[k, o],
                    device_id=((my + o) % NDEV,),
                    device_id_type=pl.DeviceIdType.MESH,
                )
                rdma.start()
                dispatch.append(rdma)

        shared = jnp.dot(xf.astype(jnp.bfloat16), sw_ref[...].astype(jnp.bfloat16),
                         preferred_element_type=jnp.float32)

        for k in range(NLOC):
            for o in range(1, NDEV):
                pltpu.make_async_remote_copy(
                    src_ref=disp_ref.at[o, pl.ds(k * CAP, CAP)],
                    dst_ref=recv_ref.at[k, o],
                    send_sem=d_send.at[k, o],
                    recv_sem=d_recv.at[k, o],
                    device_id=(my,),
                    device_id_type=pl.DeviceIdType.MESH,
                ).wait_recv()
            wk = ew_ref[k].astype(jnp.bfloat16)
            inp = recv_ref[k].reshape(NDEV * CAP, D)
            yk = jnp.dot(inp, wk, preferred_element_type=jnp.float32)
            ret_ref[k] = yk.astype(jnp.bfloat16)

        for k in range(NLOC):
            retbuf_ref[0, k * CAP:(k + 1) * CAP] = ret_ref[k, 0:CAP]

        ret = []
        for o in range(1, NDEV):
            for k in range(NLOC):
                rdma = pltpu.make_async_remote_copy(
                    src_ref=ret_ref.at[k, pl.ds(o * CAP, CAP)],
                    dst_ref=retbuf_ref.at[o, pl.ds(k * CAP, CAP)],
                    send_sem=r_send.at[k, o],
                    recv_sem=r_recv.at[k, o],
                    device_id=((my - o) % NDEV,),
                    device_id_type=pl.DeviceIdType.MESH,
                )
                rdma.start()
                ret.append(rdma)

        acc = shared
        for o in range(NDEV):
            if o > 0:
                for k in range(NLOC):
                    pltpu.make_async_remote_copy(
                        src_ref=ret_ref.at[k, pl.ds(o * CAP, CAP)],
                        dst_ref=retbuf_ref.at[o, pl.ds(k * CAP, CAP)],
                        send_sem=r_send.at[k, o],
                        recv_sem=r_recv.at[k, o],
                        device_id=(my,),
                        device_id_type=pl.DeviceIdType.MESH,
                    ).wait_recv()
            acc = acc + jnp.dot(onehot_T(o), retbuf_ref[o],
                                preferred_element_type=jnp.float32)
        out_ref[...] = acc

        for rdma in dispatch + ret:
            rdma.wait_send()

    return pl.pallas_call(
        body,
        out_shape=jax.ShapeDtypeStruct((NTOK, D), jnp.float32),
        in_specs=[pl.BlockSpec(memory_space=pltpu.VMEM)] * 5,
        out_specs=pl.BlockSpec(memory_space=pltpu.VMEM),
        scratch_shapes=[
            pltpu.VMEM((NDEV, BLK, D), jnp.bfloat16),
            pltpu.VMEM((NLOC, NDEV, CAP, D), jnp.bfloat16),
            pltpu.VMEM((NLOC, NDEV * CAP, D), jnp.bfloat16),
            pltpu.VMEM((NDEV, BLK, D), jnp.bfloat16),
            pltpu.SemaphoreType.DMA((NLOC, NDEV)),
            pltpu.SemaphoreType.DMA((NLOC, NDEV)),
            pltpu.SemaphoreType.DMA((NLOC, NDEV)),
            pltpu.SemaphoreType.DMA((NLOC, NDEV)),
        ],
        compiler_params=pltpu.CompilerParams(collective_id=0),
    )(x, router_W, route_idx, expert_W, shared_W)


# baseline (device time: 254097 ns/iter reference)
import jax
import jax.numpy as jnp
from jax import lax
from jax.experimental import pallas as pl
from jax.experimental.pallas import tpu as pltpu

NDEV = 8
NTOK = 2048
D = 1024
NLOC = 8
NEXP = NDEV * NLOC
CAP = 64
BLK = NLOC * CAP
TCH = 512


def kernel(x, router_W, route_idx, expert_W, shared_W):
    def body(x_ref, rw_ref, idx_ref, ew_ref, sw_ref, out_ref,
             disp_ref, comm2_ref, ewk_ref, xs_ref, ds_ref,
             ew_sem, ready_sem, d_send, d_recv, r_send, r_recv):
        my = lax.axis_index("i")

        barrier = pltpu.get_barrier_semaphore()
        for o in range(1, NDEV):
            pl.semaphore_signal(
                barrier, inc=1,
                device_id=((my + o) % NDEV,),
                device_id_type=pl.DeviceIdType.MESH,
            )
        pl.semaphore_wait(barrier, NDEV - 1)

        scores = jnp.dot(x_ref[...], rw_ref[...].astype(jnp.bfloat16),
                         preferred_element_type=jnp.float32)
        smax = jnp.max(scores, axis=1, keepdims=True)
        ex = jnp.exp(scores - smax)
        probs = ex / jnp.sum(ex, axis=1, keepdims=True)
        e = idx_ref[...]
        eohf = (lax.broadcasted_iota(jnp.int32, (NTOK, NEXP), 1) == e
                ).astype(jnp.float32)
        psel = jnp.sum(probs * eohf, axis=1, keepdims=True)

        c = eohf
        shift = 1
        while shift < NTOK:
            c = c + jnp.concatenate(
                [jnp.zeros((shift, NEXP), jnp.float32), c[:-shift]], axis=0)
            shift *= 2
        slot = jnp.sum((c - eohf) * eohf, axis=1, keepdims=True).astype(jnp.int32)
        dslot = jnp.where(slot < CAP, e * CAP + slot, jnp.int32(1 << 20))

        def onehot_T(o, lo):
            j = (my + o) % NDEV
            loc = ds_ref[pl.ds(lo, TCH)] - j * BLK
            return (loc == lax.broadcasted_iota(jnp.int32, (TCH, BLK), 1)
                    ).astype(jnp.bfloat16)

        ds_ref[...] = dslot
        xs_ref[...] = x_ref[...] * psel.astype(jnp.bfloat16)

        def build_o(o, carry):
            def build_h(hi, blk):
                lo = hi * TCH
                t = onehot_T(o, lo)
                xsl = xs_ref[pl.ds(lo, TCH)]
                return blk + lax.dot_general(
                    t, xsl, (((0,), (0,)), ((), ())),
                    preferred_element_type=jnp.float32)
            blk = lax.fori_loop(0, NTOK // TCH, build_h,
                                jnp.zeros((BLK, D), jnp.float32))
            disp_ref[pl.ds(o, 1)] = blk.astype(jnp.bfloat16)[None]
            return carry
        lax.fori_loop(0, NDEV, build_o, 0)

        loc_blk = disp_ref[0]
        for k in range(NLOC):
            comm2_ref[k, 0] = loc_blk[k * CAP:(k + 1) * CAP]

        dispatch = []
        for o in range(1, NDEV):
            for k in range(NLOC):
                rdma = pltpu.make_async_remote_copy(
                    src_ref=disp_ref.at[o, pl.ds(k * CAP, CAP)],
                    dst_ref=comm2_ref.at[k, o],
                    send_sem=d_send.at[k, o],
                    recv_sem=d_recv.at[k, o],
                    device_id=((my + o) % NDEV,),
                    device_id_type=pl.DeviceIdType.MESH,
                )
                rdma.start()
                dispatch.append(rdma)

        pltpu.make_async_copy(ew_ref.at[0], ewk_ref.at[0], ew_sem.at[0]).start()

        def shared_h(hi, carry):
            lo = hi * TCH
            xsl = x_ref[pl.ds(lo, TCH)]
            y = jnp.dot(xsl, sw_ref[...], preferred_element_type=jnp.float32)
            out_ref[pl.ds(lo, TCH)] = y
            return carry
        lax.fori_loop(0, NTOK // TCH, shared_h, 0)

        for rdma in dispatch:
            rdma.wait_send()

        def expert_k(k, carry):
            for o in range(1, NDEV):
                pltpu.make_async_remote_copy(
                    src_ref=disp_ref.at[o, pl.ds(0, CAP)],
                    dst_ref=comm2_ref.at[k, o],
                    send_sem=d_send.at[k, o],
                    recv_sem=d_recv.at[k, o],
                    device_id=(my,),
                    device_id_type=pl.DeviceIdType.MESH,
                ).wait_recv()
            slot2 = lax.rem(k, 2)
            pltpu.make_async_copy(
                ew_ref.at[k], ewk_ref.at[slot2], ew_sem.at[slot2]).wait()

            @pl.when(k + 1 < NLOC)
            def _prefetch():
                nslot = lax.rem(k + 1, 2)
                pltpu.make_async_copy(
                    ew_ref.at[k + 1], ewk_ref.at[nslot], ew_sem.at[nslot]
                ).start()

            inp = comm2_ref[pl.ds(k, 1)].reshape(NDEV * CAP, D)
            wk = ewk_ref[pl.ds(slot2, 1)].reshape(D, D)
            yk = jnp.dot(inp, wk, preferred_element_type=jnp.float32)
            disp_ref[pl.ds(k, 1)] = yk.astype(jnp.bfloat16)[None]
            return carry
        lax.fori_loop(0, NLOC, expert_k, 0)

        for k in range(NLOC):
            comm2_ref[0, k] = disp_ref[k, 0:CAP]
        for o in range(1, NDEV):
            pl.semaphore_signal(
                ready_sem, inc=1,
                device_id=((my + o) % NDEV,),
                device_id_type=pl.DeviceIdType.MESH,
            )
        pl.semaphore_wait(ready_sem, NDEV - 1)

        ret = []
        for o in range(1, NDEV):
            for k in range(NLOC):
                rdma = pltpu.make_async_remote_copy(
                    src_ref=disp_ref.at[k, pl.ds(o * CAP, CAP)],
                    dst_ref=comm2_ref.at[o, k],
                    send_sem=r_send.at[k, o],
                    recv_sem=r_recv.at[k, o],
                    device_id=((my - o) % NDEV,),
                    device_id_type=pl.DeviceIdType.MESH,
                )
                rdma.start()
                ret.append(rdma)

        for o in range(1, NDEV):
            for k in range(NLOC):
                pltpu.make_async_remote_copy(
                    src_ref=disp_ref.at[k, pl.ds(o * CAP, CAP)],
                    dst_ref=comm2_ref.at[o, k],
                    send_sem=r_send.at[k, o],
                    recv_sem=r_recv.at[k, o],
                    device_id=(my,),
                    device_id_type=pl.DeviceIdType.MESH,
                ).wait_recv()

        def comb_o(o, carry):
            rb = comm2_ref[pl.ds(o, 1)].reshape(BLK, D)

            def comb_h(hi, c2):
                lo = hi * TCH
                t = onehot_T(o, lo)
                cur = out_ref[pl.ds(lo, TCH)]
                out_ref[pl.ds(lo, TCH)] = cur + jnp.dot(
                    t, rb, preferred_element_type=jnp.float32)
                return c2
            return lax.fori_loop(0, NTOK // TCH, comb_h, carry)
        lax.fori_loop(0, NDEV, comb_o, 0)

        for rdma in ret:
            rdma.wait_send()

    xb = x.astype(jnp.bfloat16)
    ewb = expert_W.astype(jnp.bfloat16)
    swb = shared_W.astype(jnp.bfloat16)
    return pl.pallas_call(
        body,
        out_shape=jax.ShapeDtypeStruct((NTOK, D), jnp.float32),
        in_specs=[
            pl.BlockSpec(memory_space=pltpu.VMEM),
            pl.BlockSpec(memory_space=pltpu.VMEM),
            pl.BlockSpec(memory_space=pltpu.VMEM),
            pl.BlockSpec(memory_space=pl.ANY),
            pl.BlockSpec(memory_space=pltpu.VMEM),
        ],
        out_specs=pl.BlockSpec(memory_space=pltpu.VMEM),
        scratch_shapes=[
            pltpu.VMEM((NDEV, BLK, D), jnp.bfloat16),
            pltpu.VMEM((NDEV, NDEV, CAP, D), jnp.bfloat16),
            pltpu.VMEM((2, D, D), jnp.bfloat16),
            pltpu.VMEM((NTOK, D), jnp.bfloat16),
            pltpu.VMEM((NTOK, 1), jnp.int32),
            pltpu.SemaphoreType.DMA((2,)),
            pltpu.SemaphoreType.REGULAR,
            pltpu.SemaphoreType.DMA((NLOC, NDEV)),
            pltpu.SemaphoreType.DMA((NLOC, NDEV)),
            pltpu.SemaphoreType.DMA((NLOC, NDEV)),
            pltpu.SemaphoreType.DMA((NLOC, NDEV)),
        ],
        compiler_params=pltpu.CompilerParams(
            collective_id=0, vmem_limit_bytes=50 * 1024 * 1024),
    )(xb, router_W, route_idx, ewb, swb)


# device time: 204380 ns/iter; 1.2433x vs baseline; 1.2433x over previous
import jax
import jax.numpy as jnp
from jax import lax
from jax.experimental import pallas as pl
from jax.experimental.pallas import tpu as pltpu

NDEV = 8
NTOK = 2048
D = 1024
NLOC = 8
NEXP = NDEV * NLOC
CAP = 64
BLK = NLOC * CAP
TCH = 512


def kernel(x, router_W, route_idx, expert_W, shared_W):
    def body(x_ref, rw_ref, idx_ref, ew_ref, sw_ref, out_ref,
             disp_ref, comm2_ref, ewk_ref, xs_ref, ds_ref,
             ew_sem, ready_sem, d_send, d_recv, r_send, r_recv):
        my = lax.axis_index("i")

        barrier = pltpu.get_barrier_semaphore()
        for o in range(1, NDEV):
            pl.semaphore_signal(
                barrier, inc=1,
                device_id=((my + o) % NDEV,),
                device_id_type=pl.DeviceIdType.MESH,
            )
        pl.semaphore_wait(barrier, NDEV - 1)

        scores = jnp.dot(x_ref[...], rw_ref[...].astype(jnp.bfloat16),
                         preferred_element_type=jnp.float32)
        smax = jnp.max(scores, axis=1, keepdims=True)
        ex = jnp.exp(scores - smax)
        probs = ex / jnp.sum(ex, axis=1, keepdims=True)
        e = idx_ref[...]
        eohf = (lax.broadcasted_iota(jnp.int32, (NTOK, NEXP), 1) == e
                ).astype(jnp.float32)
        psel = jnp.sum(probs * eohf, axis=1, keepdims=True)

        c = eohf
        shift = 1
        while shift < NTOK:
            c = c + jnp.concatenate(
                [jnp.zeros((shift, NEXP), jnp.float32), c[:-shift]], axis=0)
            shift *= 2
        slot = jnp.sum((c - eohf) * eohf, axis=1, keepdims=True).astype(jnp.int32)
        dslot = jnp.where(slot < CAP, e * CAP + slot, jnp.int32(1 << 20))

        def onehot_T(o, lo):
            j = (my + o) % NDEV
            loc = ds_ref[pl.ds(lo, TCH)] - j * BLK
            return (loc == lax.broadcasted_iota(jnp.int32, (TCH, BLK), 1)
                    ).astype(jnp.bfloat16)

        pltpu.make_async_copy(ew_ref.at[0], ewk_ref.at[0], ew_sem.at[0]).start()

        ds_ref[...] = dslot
        xs_ref[...] = x_ref[...] * psel.astype(jnp.bfloat16)

        def build_o(o, carry):
            def build_h(hi, blk):
                lo = hi * TCH
                t = onehot_T(o, lo)
                xsl = xs_ref[pl.ds(lo, TCH)]
                return blk + lax.dot_general(
                    t, xsl, (((0,), (0,)), ((), ())),
                    preferred_element_type=jnp.float32)
            blk = lax.fori_loop(0, NTOK // TCH, build_h,
                                jnp.zeros((BLK, D), jnp.float32))
            disp_ref[pl.ds(o, 1)] = blk.astype(jnp.bfloat16)[None]

            @pl.when(o > 0)
            def _send():
                for k in range(NLOC):
                    pltpu.make_async_remote_copy(
                        src_ref=disp_ref.at[o, pl.ds(k * CAP, CAP)],
                        dst_ref=comm2_ref.at[k, o],
                        send_sem=d_send.at[k, o],
                        recv_sem=d_recv.at[k, o],
                        device_id=((my + o) % NDEV,),
                        device_id_type=pl.DeviceIdType.MESH,
                    ).start()
            return carry
        lax.fori_loop(0, NDEV, build_o, 0)

        loc_blk = disp_ref[0]
        for k in range(NLOC):
            comm2_ref[k, 0] = loc_blk[k * CAP:(k + 1) * CAP]

        def shared_h(hi, carry):
            lo = hi * TCH
            xsl = x_ref[pl.ds(lo, TCH)]
            y = jnp.dot(xsl, sw_ref[...], preferred_element_type=jnp.float32)
            out_ref[pl.ds(lo, TCH)] = y
            return carry
        lax.fori_loop(0, NTOK // TCH, shared_h, 0)

        for o in range(1, NDEV):
            for k in range(NLOC):
                pltpu.make_async_remote_copy(
                    src_ref=disp_ref.at[o, pl.ds(k * CAP, CAP)],
                    dst_ref=comm2_ref.at[k, o],
                    send_sem=d_send.at[k, o],
                    recv_sem=d_recv.at[k, o],
                    device_id=(my,),
                    device_id_type=pl.DeviceIdType.MESH,
                ).wait_send()

        def expert_k(k, carry):
            for o in range(1, NDEV):
                pltpu.make_async_remote_copy(
                    src_ref=disp_ref.at[o, pl.ds(0, CAP)],
                    dst_ref=comm2_ref.at[k, o],
                    send_sem=d_send.at[k, o],
                    recv_sem=d_recv.at[k, o],
                    device_id=(my,),
                    device_id_type=pl.DeviceIdType.MESH,
                ).wait_recv()
            slot2 = lax.rem(k, 2)
            pltpu.make_async_copy(
                ew_ref.at[k], ewk_ref.at[slot2], ew_sem.at[slot2]).wait()

            @pl.when(k + 1 < NLOC)
            def _prefetch():
                nslot = lax.rem(k + 1, 2)
                pltpu.make_async_copy(
                    ew_ref.at[k + 1], ewk_ref.at[nslot], ew_sem.at[nslot]
                ).start()

            inp = comm2_ref[pl.ds(k, 1)].reshape(NDEV * CAP, D)
            wk = ewk_ref[pl.ds(slot2, 1)].reshape(D, D)
            yk = jnp.dot(inp, wk, preferred_element_type=jnp.float32)
            disp_ref[pl.ds(k, 1)] = yk.astype(jnp.bfloat16)[None]
            return carry
        lax.fori_loop(0, NLOC, expert_k, 0)

        for k in range(NLOC):
            comm2_ref[0, k] = disp_ref[k, 0:CAP]
        for o in range(1, NDEV):
            pl.semaphore_signal(
                ready_sem, inc=1,
                device_id=((my + o) % NDEV,),
                device_id_type=pl.DeviceIdType.MESH,
            )
        pl.semaphore_wait(ready_sem, NDEV - 1)

        ret = []
        for o in range(1, NDEV):
            for k in range(NLOC):
                rdma = pltpu.make_async_remote_copy(
                    src_ref=disp_ref.at[k, pl.ds(o * CAP, CAP)],
                    dst_ref=comm2_ref.at[o, k],
                    send_sem=r_send.at[k, o],
                    recv_sem=r_recv.at[k, o],
                    device_id=((my - o) % NDEV,),
                    device_id_type=pl.DeviceIdType.MESH,
                )
                rdma.start()
                ret.append(rdma)

        def comb_o(o, carry):
            @pl.when(o > 0)
            def _wait():
                for k in range(NLOC):
                    pltpu.make_async_remote_copy(
                        src_ref=disp_ref.at[k, pl.ds(0, CAP)],
                        dst_ref=comm2_ref.at[o, k],
                        send_sem=r_send.at[k, o],
                        recv_sem=r_recv.at[k, o],
                        device_id=(my,),
                        device_id_type=pl.DeviceIdType.MESH,
                    ).wait_recv()
            rb = comm2_ref[pl.ds(o, 1)].reshape(BLK, D)

            def comb_h(hi, c2):
                lo = hi * TCH
                t = onehot_T(o, lo)
                cur = out_ref[pl.ds(lo, TCH)]
                out_ref[pl.ds(lo, TCH)] = cur + jnp.dot(
                    t, rb, preferred_element_type=jnp.float32)
                return c2
            return lax.fori_loop(0, NTOK // TCH, comb_h, carry)
        lax.fori_loop(0, NDEV, comb_o, 0)

        for rdma in ret:
            rdma.wait_send()

    xb = x.astype(jnp.bfloat16)
    ewb = expert_W.astype(jnp.bfloat16)
    swb = shared_W.astype(jnp.bfloat16)
    return pl.pallas_call(
        body,
        out_shape=jax.ShapeDtypeStruct((NTOK, D), jnp.float32),
        in_specs=[
            pl.BlockSpec(memory_space=pltpu.VMEM),
            pl.BlockSpec(memory_space=pltpu.VMEM),
            pl.BlockSpec(memory_space=pltpu.VMEM),
            pl.BlockSpec(memory_space=pl.ANY),
            pl.BlockSpec(memory_space=pltpu.VMEM),
        ],
        out_specs=pl.BlockSpec(memory_space=pltpu.VMEM),
        scratch_shapes=[
            pltpu.VMEM((NDEV, BLK, D), jnp.bfloat16),
            pltpu.VMEM((NDEV, NDEV, CAP, D), jnp.bfloat16),
            pltpu.VMEM((2, D, D), jnp.bfloat16),
            pltpu.VMEM((NTOK, D), jnp.bfloat16),
            pltpu.VMEM((NTOK, 1), jnp.int32),
            pltpu.SemaphoreType.DMA((2,)),
            pltpu.SemaphoreType.REGULAR,
            pltpu.SemaphoreType.DMA((NLOC, NDEV)),
            pltpu.SemaphoreType.DMA((NLOC, NDEV)),
            pltpu.SemaphoreType.DMA((NLOC, NDEV)),
            pltpu.SemaphoreType.DMA((NLOC, NDEV)),
        ],
        compiler_params=pltpu.CompilerParams(
            collective_id=0, vmem_limit_bytes=50 * 1024 * 1024),
    )(xb, router_W, route_idx, ewb, swb)
